# baseline (speedup 1.0000x reference)
import functools
import math

import jax
import jax.numpy as jnp
from jax import lax
from jax.experimental import pallas as pl
from jax.experimental.pallas import tpu as pltpu
from jax.experimental.pallas import tpu_sc as plsc

D_MODEL = 64
SCALE = math.sqrt(D_MODEL)
NUM_CORES = 2
NUM_SUBCORES = 16
NUM_WORKERS = NUM_CORES * NUM_SUBCORES
VOCAB = 1000000

VB = 256
TAIL_BLOCK = 64
TAIL_START = (VOCAB // (VB * NUM_WORKERS)) * VB * NUM_WORKERS
TAIL_WORKERS = (VOCAB - TAIL_START) // TAIL_BLOCK


def _prep(table_t):
    mesh = plsc.VectorSubcoreMesh(core_axis_name="c", subcore_axis_name="s")
    n_blocks = VOCAB // VB
    blocks_per_w = n_blocks // NUM_WORKERS

    @functools.partial(
        pl.kernel,
        out_type=jax.ShapeDtypeStruct((VOCAB, D_MODEL), jnp.float32),
        mesh=mesh,
        scratch_types=[
            pltpu.VMEM((2, D_MODEL, VB), jnp.float32),
            pltpu.VMEM((2, VB, D_MODEL), jnp.float32),
        ]
        + [pltpu.SemaphoreType.DMA] * 4,
        compiler_params=pltpu.CompilerParams(use_tc_tiling_on_sc=False, needs_layout_passes=False),
    )
    def body(tab_t, tab_r, in_v, out_v, gs0, gs1, ss0, ss1):
        gsem = (gs0, gs1)
        ssem = (ss0, ss1)
        wid = lax.axis_index("s") * NUM_CORES + lax.axis_index("c")
        m0 = wid * blocks_per_w

        def issue_in(m, slot):
            pltpu.async_copy(
                tab_t.at[:, pl.ds((m0 + m) * VB, VB)], in_v.at[slot], gsem[slot]
            )

        issue_in(0, 0)

        def block_body(m, carry):
            for slot in range(2):
                bm = m * 2 + slot
                pltpu.make_async_copy(
                    tab_t.at[:, pl.ds(0, VB)], in_v.at[slot], gsem[slot]
                ).wait()

                @pl.when(bm + 1 < blocks_per_w)
                def _():
                    issue_in(bm + 1, 1 - slot)

                @pl.when(bm >= 2)
                def _():
                    pltpu.make_async_copy(
                        out_v.at[slot], tab_r.at[pl.ds(0, VB)], ssem[slot]
                    ).wait()

                @plsc.parallel_loop(0, VB, 1, unroll=4)
                def _(r):
                    cols = jnp.full((16,), r, jnp.int32)
                    for q in range(D_MODEL // 16):
                        rows = jax.lax.iota(jnp.int32, 16) + (q * 16)
                        vals = plsc.load_gather(in_v.at[slot], [rows, cols])
                        out_v[slot, r, pl.ds(q * 16, 16)] = vals * SCALE

                pltpu.async_copy(
                    out_v.at[slot],
                    tab_r.at[pl.ds((m0 + bm) * VB, VB)],
                    ssem[slot],
                )
            return carry

        lax.fori_loop(0, blocks_per_w // 2, block_body, 0)
        for slot in range(2):
            pltpu.make_async_copy(
                out_v.at[slot], tab_r.at[pl.ds(0, VB)], ssem[slot]
            ).wait()

        @pl.when(wid < TAIL_WORKERS)
        def _():
            off = TAIL_START + wid * TAIL_BLOCK
            pltpu.sync_copy(
                tab_t.at[:, pl.ds(off, TAIL_BLOCK)],
                in_v.at[0, :, pl.ds(0, TAIL_BLOCK)],
            )

            @plsc.parallel_loop(0, TAIL_BLOCK, 1, unroll=4)
            def _(r):
                cols = jnp.full((16,), r, jnp.int32)
                for q in range(D_MODEL // 16):
                    rows = jax.lax.iota(jnp.int32, 16) + (q * 16)
                    vals = plsc.load_gather(in_v.at[0], [rows, cols])
                    out_v[0, r, pl.ds(q * 16, 16)] = vals * SCALE

            pltpu.sync_copy(
                out_v.at[0, pl.ds(0, TAIL_BLOCK)],
                tab_r.at[pl.ds(off, TAIL_BLOCK)],
            )

    return body(table_t)


CHUNK = 128
NBUF = 4
AHEAD = NBUF - 1


def _gather(tokens_flat, table_r, B, S, SEQ):
    mesh = plsc.VectorSubcoreMesh(core_axis_name="c", subcore_axis_name="s")
    b_per_w = B // NUM_WORKERS
    n_chunks = b_per_w // CHUNK

    @functools.partial(
        pl.kernel,
        out_type=jax.ShapeDtypeStruct((SEQ, D_MODEL, S), jnp.float32),
        mesh=mesh,
        scratch_types=[
            pltpu.VMEM((NBUF, CHUNK), jnp.int32),
            pltpu.VMEM((NBUF, CHUNK, D_MODEL), jnp.float32),
            pltpu.VMEM((NBUF, D_MODEL, CHUNK), jnp.float32),
        ]
        + [pltpu.SemaphoreType.DMA] * (2 * NBUF),
        compiler_params=pltpu.CompilerParams(use_tc_tiling_on_sc=False, needs_layout_passes=False),
    )
    def body(tok_hbm, tab_hbm, out_hbm, idx_v, rows_v, obuf_v, *sems):
        gsem = sems[:NBUF]
        ssem = sems[NBUF:]
        wid = lax.axis_index("s") * NUM_CORES + lax.axis_index("c")
        base = wid * b_per_w

        def issue_gather(g, slot):
            off = base + g * CHUNK
            pltpu.sync_copy(tok_hbm.at[pl.ds(off, CHUNK)], idx_v.at[slot])
            pltpu.async_copy(
                tab_hbm.at[idx_v.at[slot]], rows_v.at[slot], gsem[slot]
            )

        for g in range(AHEAD):
            issue_gather(g, g % NBUF)

        def outer(t, carry):
            for j in range(NBUF):
                g = t * NBUF + j
                pltpu.make_async_copy(
                    tab_hbm.at[idx_v.at[j]], rows_v.at[j], gsem[j]
                ).wait()

                @pl.when(g >= NBUF)
                def _():
                    pltpu.make_async_copy(
                        obuf_v.at[j],
                        out_hbm.at[0, :, pl.ds(0, CHUNK)],
                        ssem[j],
                    ).wait()

                @plsc.parallel_loop(0, D_MODEL, 1, unroll=4)
                def _(d):
                    cols = jnp.full((16,), d, jnp.int32)
                    for q in range(CHUNK // 16):
                        rows16 = jax.lax.iota(jnp.int32, 16) + (q * 16)
                        vals = plsc.load_gather(rows_v.at[j], [rows16, cols])
                        obuf_v[j, d, pl.ds(q * 16, 16)] = vals

                off = base + g * CHUNK
                s = off // S
                b0 = off - s * S
                pltpu.async_copy(
                    obuf_v.at[j],
                    out_hbm.at[s, :, pl.ds(b0, CHUNK)],
                    ssem[j],
                )

                nxt = g + AHEAD

                @pl.when(nxt < n_chunks)
                def _():
                    issue_gather(nxt, (j + AHEAD) % NBUF)

            return carry

        lax.fori_loop(0, n_chunks // NBUF, outer, 0)

        for j in range(NBUF):
            pltpu.make_async_copy(
                obuf_v.at[j], out_hbm.at[0, :, pl.ds(0, CHUNK)], ssem[j]
            ).wait()

    return body(tokens_flat, table_r)


def kernel(tokens, table):
    S, SEQ = tokens.shape
    B = S * SEQ
    tok_flat = jnp.transpose(tokens).reshape(B).astype(jnp.int32)
    table_t = jnp.transpose(table)
    table_r = _prep(table_t)
    out_t = _gather(tok_flat, table_r, B, S, SEQ)
    return jnp.transpose(out_t, (2, 0, 1))


# speedup vs baseline: 5.2838x; 5.2838x over previous
import functools
import math

import jax
import jax.numpy as jnp
from jax import lax
from jax.experimental import pallas as pl
from jax.experimental.pallas import tpu as pltpu
from jax.experimental.pallas import tpu_sc as plsc

D_MODEL = 64
SCALE = math.sqrt(D_MODEL)
NUM_CORES = 2
NUM_SUBCORES = 16
NUM_WORKERS = NUM_CORES * NUM_SUBCORES

CHUNK = 128
NBUF = 4
AHEAD = NBUF - 1


def _gather(tokens_flat, table, B):
    mesh = plsc.VectorSubcoreMesh(core_axis_name="c", subcore_axis_name="s")
    b_per_w = B // NUM_WORKERS
    n_chunks = b_per_w // CHUNK

    @functools.partial(
        pl.kernel,
        out_type=jax.ShapeDtypeStruct((B, D_MODEL), jnp.float32),
        mesh=mesh,
        scratch_types=[
            pltpu.VMEM((NBUF, CHUNK), jnp.int32),
            pltpu.VMEM((NBUF, CHUNK, D_MODEL), jnp.float32),
        ]
        + [pltpu.SemaphoreType.DMA] * (2 * NBUF),
        compiler_params=pltpu.CompilerParams(
            use_tc_tiling_on_sc=False, needs_layout_passes=False
        ),
    )
    def body(tok_hbm, tab_hbm, out_hbm, idx_v, rows_v, *sems):
        gsem = sems[:NBUF]
        ssem = sems[NBUF:]
        wid = lax.axis_index("s") * NUM_CORES + lax.axis_index("c")
        base = wid * b_per_w

        def issue_gather(g, slot):
            off = base + g * CHUNK
            pltpu.sync_copy(tok_hbm.at[pl.ds(off, CHUNK)], idx_v.at[slot])
            pltpu.async_copy(
                tab_hbm.at[idx_v.at[slot]], rows_v.at[slot], gsem[slot]
            )

        for g in range(AHEAD):
            issue_gather(g, g % NBUF)

        def outer(t, carry):
            for j in range(NBUF):
                g = t * NBUF + j
                pltpu.make_async_copy(
                    tab_hbm.at[idx_v.at[j]], rows_v.at[j], gsem[j]
                ).wait()

                @plsc.parallel_loop(0, CHUNK, 1, unroll=4)
                def _(r):
                    for q in range(D_MODEL // 16):
                        sl = pl.ds(q * 16, 16)
                        rows_v[j, r, sl] = rows_v[j, r, sl] * SCALE

                off = base + g * CHUNK
                pltpu.async_copy(
                    rows_v.at[j], out_hbm.at[pl.ds(off, CHUNK)], ssem[j]
                )

                nxt = g + AHEAD

                @pl.when(nxt < n_chunks)
                def _():
                    slot = (j + AHEAD) % NBUF
                    @pl.when(nxt >= NBUF)
                    def _():
                        pltpu.make_async_copy(
                            rows_v.at[slot],
                            out_hbm.at[pl.ds(0, CHUNK)],
                            ssem[slot],
                        ).wait()

                    issue_gather(nxt, slot)

            return carry

        lax.fori_loop(0, n_chunks // NBUF, outer, 0)

        for j in range(NBUF):
            pltpu.make_async_copy(
                rows_v.at[j], out_hbm.at[pl.ds(0, CHUNK)], ssem[j]
            ).wait()

    return body(tokens_flat, table)


def kernel(tokens, table):
    S, SEQ = tokens.shape
    B = S * SEQ
    tok_flat = tokens.reshape(B).astype(jnp.int32)
    out = _gather(tok_flat, table, B)
    return out.reshape(S, SEQ, D_MODEL)
